# baseline (speedup 1.0000x reference)
import jax
import jax.numpy as jnp
from jax.experimental import pallas as pl
from jax.experimental.pallas import tpu as pltpu

EPS = 1e-5
BF16 = jnp.bfloat16


def _zero_border(pad, nb, hp, wp, c):
    zrow = jnp.zeros((nb, 1, wp + 2, c), BF16)
    zcol = jnp.zeros((nb, hp + 2, 1, c), BF16)
    pad[:, 0:1, :, :] = zrow
    pad[:, hp + 1:hp + 2, :, :] = zrow
    pad[:, :, 0:1, :] = zcol
    pad[:, :, wp + 1:wp + 2, :] = zcol


def _conv3x3(pad, w_ref, nb, hp, wp, cin):
    cols = jnp.concatenate(
        [pad[:, dy:dy + hp, dx:dx + wp, :] for dy in range(3) for dx in range(3)],
        axis=-1)
    a = cols.reshape(nb * hp * wp, 9 * cin)
    return jnp.dot(a, w_ref[...], preferred_element_type=jnp.float32)


def _emit_stats(y, st_ref, cout):
    s = jnp.sum(y, axis=0, keepdims=True)
    ss = jnp.sum(y * y, axis=0, keepdims=True)
    st_ref[...] = jnp.concatenate([s, ss], axis=0).reshape(1, 2, cout)


def _stage1_kernel(x_ref, w_ref, y_ref, st_ref, pad):
    nb, hp, wp, c4 = x_ref.shape
    c = c4 // 4
    cout = w_ref.shape[1]

    @pl.when(pl.program_id(0) == 0)
    def _():
        _zero_border(pad, nb, hp, wp, c)

    xv = x_ref[...]
    pooled = jnp.maximum(jnp.maximum(xv[..., :c], xv[..., c:2 * c]),
                         jnp.maximum(xv[..., 2 * c:3 * c], xv[..., 3 * c:]))
    pad[:, 1:hp + 1, 1:wp + 1, :] = pooled
    y = _conv3x3(pad, w_ref, nb, hp, wp, c)
    _emit_stats(y, st_ref, cout)
    y_ref[...] = y.astype(BF16).reshape(nb, hp, wp, cout)


def _make_stage2_kernel(inv_npix):
    def _kernel(y1_ref, st1_ref, g_ref, be_ref, w_ref, y_ref, st_ref, pad):
        nb, hp, wp, c = y1_ref.shape
        cout = w_ref.shape[1]

        @pl.when(pl.program_id(0) == 0)
        def _():
            _zero_border(pad, nb, hp, wp, c)

        st = st1_ref[...]
        s = jnp.sum(st[:, 0:1, :], axis=0)
        ss = jnp.sum(st[:, 1:2, :], axis=0)
        mean = s * inv_npix
        var = ss * inv_npix - mean * mean
        scale = g_ref[...] * jax.lax.rsqrt(var + EPS)
        shift = be_ref[...] - mean * scale

        h = jnp.maximum(y1_ref[...].astype(jnp.float32) * scale.reshape(1, 1, 1, c)
                        + shift.reshape(1, 1, 1, c), 0.0)
        pad[:, 1:hp + 1, 1:wp + 1, :] = h.astype(BF16)
        y = _conv3x3(pad, w_ref, nb, hp, wp, c)
        _emit_stats(y, st_ref, cout)
        y_ref[...] = y.astype(BF16).reshape(nb, hp, wp, cout)
    return _kernel


def _pick_images_per_block(n, hp, wp, c):
    budget = 24 * 1024 * 1024
    nb = 1
    for d in range(1, n + 1):
        if n % d:
            continue
        m = d * hp * wp
        need = 2 * (9 * m * c + d * (hp + 2) * (wp + 2) * c + 4 * m * c) + 8 * m * c
        if need > budget:
            break
        nb = d
    return nb


def kernel(x_nchw, w1, b1, g1, be1, w2, b2, g2, be2):
    N, Cin, H, W = x_nchw.shape
    Hp, Wp = H // 2, W // 2
    Cout = w1.shape[1]
    n_pix = float(N * Hp * Wp)

    nb = _pick_images_per_block(N, Hp, Wp, Cout)
    n_blk = N // nb

    x6 = x_nchw[:, :, :2 * Hp, :2 * Wp].reshape(N, Cin, Hp, 2, Wp, 2)
    xt = jnp.transpose(x6, (0, 2, 4, 3, 5, 1)).astype(BF16)
    x5 = xt.reshape(N, Hp, Wp, 4 * Cin)
    w1b = w1.astype(BF16)
    w2b = w2.astype(BF16)

    cparams = pltpu.CompilerParams(
        dimension_semantics=("arbitrary",),
        vmem_limit_bytes=48 * 1024 * 1024)

    y1, st1 = pl.pallas_call(
        _stage1_kernel,
        grid=(n_blk,),
        in_specs=[
            pl.BlockSpec((nb, Hp, Wp, 4 * Cin), lambda i: (i, 0, 0, 0)),
            pl.BlockSpec((9 * Cin, Cout), lambda i: (0, 0)),
        ],
        out_specs=(
            pl.BlockSpec((nb, Hp, Wp, Cout), lambda i: (i, 0, 0, 0)),
            pl.BlockSpec((1, 2, Cout), lambda i: (i, 0, 0)),
        ),
        out_shape=(
            jax.ShapeDtypeStruct((N, Hp, Wp, Cout), BF16),
            jax.ShapeDtypeStruct((n_blk, 2, Cout), jnp.float32),
        ),
        scratch_shapes=[pltpu.VMEM((nb, Hp + 2, Wp + 2, Cin), BF16)],
        compiler_params=cparams,
    )(x5, w1b)

    y2, st2 = pl.pallas_call(
        _make_stage2_kernel(1.0 / n_pix),
        grid=(n_blk,),
        in_specs=[
            pl.BlockSpec((nb, Hp, Wp, Cout), lambda i: (i, 0, 0, 0)),
            pl.BlockSpec((n_blk, 2, Cout), lambda i: (0, 0, 0)),
            pl.BlockSpec((1, Cout), lambda i: (0, 0)),
            pl.BlockSpec((1, Cout), lambda i: (0, 0)),
            pl.BlockSpec((9 * Cout, Cout), lambda i: (0, 0)),
        ],
        out_specs=(
            pl.BlockSpec((nb, Hp, Wp, Cout), lambda i: (i, 0, 0, 0)),
            pl.BlockSpec((1, 2, Cout), lambda i: (i, 0, 0)),
        ),
        out_shape=(
            jax.ShapeDtypeStruct((N, Hp, Wp, Cout), BF16),
            jax.ShapeDtypeStruct((n_blk, 2, Cout), jnp.float32),
        ),
        scratch_shapes=[pltpu.VMEM((nb, Hp + 2, Wp + 2, Cout), BF16)],
        compiler_params=cparams,
    )(y1, st1, g1, be1, w2b)

    s = jnp.sum(st2[:, 0, :], axis=0)
    ss = jnp.sum(st2[:, 1, :], axis=0)
    mean = s / n_pix
    var = ss / n_pix - mean * mean
    scale = g2[0] * jax.lax.rsqrt(var + EPS)
    shift = be2[0] - mean * scale
    out = jnp.maximum(y2.astype(jnp.float32) * scale.reshape(1, 1, 1, Cout)
                      + shift.reshape(1, 1, 1, Cout), 0.0)
    return jnp.transpose(out, (0, 3, 1, 2))


# speedup vs baseline: 1.6719x; 1.6719x over previous
import jax
import jax.numpy as jnp
from jax.experimental import pallas as pl
from jax.experimental.pallas import tpu as pltpu

EPS = 1e-5
BF16 = jnp.bfloat16


def _zero_border(pad, nb, hp, wp, c):
    zrow = jnp.zeros((nb, 1, wp + 2, c), BF16)
    zcol = jnp.zeros((nb, hp + 2, 1, c), BF16)
    pad[:, 0:1, :, :] = zrow
    pad[:, hp + 1:hp + 2, :, :] = zrow
    pad[:, :, 0:1, :] = zcol
    pad[:, :, wp + 1:wp + 2, :] = zcol


def _conv3x3(pad, w_ref, nb, hp, wp, cin):
    cols = jnp.concatenate(
        [pad[:, dy:dy + hp, dx:dx + wp, :] for dy in range(3) for dx in range(3)],
        axis=-1)
    a = cols.reshape(nb * hp * wp, 9 * cin)
    return jnp.dot(a, w_ref[...], preferred_element_type=jnp.float32)


def _emit_stats(y, st_ref, cout):
    s = jnp.sum(y, axis=0, keepdims=True)
    ss = jnp.sum(y * y, axis=0, keepdims=True)
    st_ref[...] = jnp.concatenate([s, ss], axis=0).reshape(1, 2, cout)


def _stage1_kernel(x_ref, w_ref, y_ref, st_ref, pad):
    nb, hp, _, wp, c2 = x_ref.shape
    c = c2 // 2
    cout = w_ref.shape[1]

    @pl.when(pl.program_id(0) == 0)
    def _():
        _zero_border(pad, nb, hp, wp, c)

    xv = x_ref[...]
    rows = jnp.maximum(xv[:, :, 0, :, :], xv[:, :, 1, :, :])
    pooled = jnp.maximum(rows[..., :c], rows[..., c:])
    pad[:, 1:hp + 1, 1:wp + 1, :] = pooled
    y = _conv3x3(pad, w_ref, nb, hp, wp, c)
    _emit_stats(y, st_ref, cout)
    y_ref[...] = y.astype(BF16).reshape(nb, hp, wp, cout)


def _make_stage2_kernel(inv_npix):
    def _kernel(y1_ref, st1_ref, g_ref, be_ref, w_ref, y_ref, st_ref, pad):
        nb, hp, wp, c = y1_ref.shape
        cout = w_ref.shape[1]

        @pl.when(pl.program_id(0) == 0)
        def _():
            _zero_border(pad, nb, hp, wp, c)

        st = st1_ref[...]
        s = jnp.sum(st[:, 0:1, :], axis=0)
        ss = jnp.sum(st[:, 1:2, :], axis=0)
        mean = s * inv_npix
        var = ss * inv_npix - mean * mean
        scale = g_ref[...] * jax.lax.rsqrt(var + EPS)
        shift = be_ref[...] - mean * scale

        h = jnp.maximum(y1_ref[...].astype(jnp.float32) * scale.reshape(1, 1, 1, c)
                        + shift.reshape(1, 1, 1, c), 0.0)
        pad[:, 1:hp + 1, 1:wp + 1, :] = h.astype(BF16)
        y = _conv3x3(pad, w_ref, nb, hp, wp, c)
        _emit_stats(y, st_ref, cout)
        y_ref[...] = y.astype(BF16).reshape(nb, hp, wp, cout)
    return _kernel


def _pick_images_per_block(n, hp, wp, c):
    budget = 24 * 1024 * 1024
    nb = 1
    for d in range(1, n + 1):
        if n % d:
            continue
        m = d * hp * wp
        need = 2 * (9 * m * c + d * (hp + 2) * (wp + 2) * c + 4 * m * c) + 8 * m * c
        if need > budget:
            break
        nb = d
    return nb


def kernel(x_nchw, w1, b1, g1, be1, w2, b2, g2, be2):
    N, Cin, H, W = x_nchw.shape
    Hp, Wp = H // 2, W // 2
    Cout = w1.shape[1]
    n_pix = float(N * Hp * Wp)

    nb = _pick_images_per_block(N, Hp, Wp, Cout)
    n_blk = N // nb

    xb = x_nchw[:, :, :2 * Hp, :2 * Wp].astype(BF16)
    x = jnp.transpose(xb, (0, 2, 3, 1))
    x5 = x.reshape(N, Hp, 2, Wp, 2 * Cin)
    w1b = w1.astype(BF16)
    w2b = w2.astype(BF16)

    cparams = pltpu.CompilerParams(
        dimension_semantics=("arbitrary",),
        vmem_limit_bytes=48 * 1024 * 1024)

    y1, st1 = pl.pallas_call(
        _stage1_kernel,
        grid=(n_blk,),
        in_specs=[
            pl.BlockSpec((nb, Hp, 2, Wp, 2 * Cin), lambda i: (i, 0, 0, 0, 0)),
            pl.BlockSpec((9 * Cin, Cout), lambda i: (0, 0)),
        ],
        out_specs=(
            pl.BlockSpec((nb, Hp, Wp, Cout), lambda i: (i, 0, 0, 0)),
            pl.BlockSpec((1, 2, Cout), lambda i: (i, 0, 0)),
        ),
        out_shape=(
            jax.ShapeDtypeStruct((N, Hp, Wp, Cout), BF16),
            jax.ShapeDtypeStruct((n_blk, 2, Cout), jnp.float32),
        ),
        scratch_shapes=[pltpu.VMEM((nb, Hp + 2, Wp + 2, Cin), BF16)],
        compiler_params=cparams,
    )(x5, w1b)

    y2, st2 = pl.pallas_call(
        _make_stage2_kernel(1.0 / n_pix),
        grid=(n_blk,),
        in_specs=[
            pl.BlockSpec((nb, Hp, Wp, Cout), lambda i: (i, 0, 0, 0)),
            pl.BlockSpec((n_blk, 2, Cout), lambda i: (0, 0, 0)),
            pl.BlockSpec((1, Cout), lambda i: (0, 0)),
            pl.BlockSpec((1, Cout), lambda i: (0, 0)),
            pl.BlockSpec((9 * Cout, Cout), lambda i: (0, 0)),
        ],
        out_specs=(
            pl.BlockSpec((nb, Hp, Wp, Cout), lambda i: (i, 0, 0, 0)),
            pl.BlockSpec((1, 2, Cout), lambda i: (i, 0, 0)),
        ),
        out_shape=(
            jax.ShapeDtypeStruct((N, Hp, Wp, Cout), BF16),
            jax.ShapeDtypeStruct((n_blk, 2, Cout), jnp.float32),
        ),
        scratch_shapes=[pltpu.VMEM((nb, Hp + 2, Wp + 2, Cout), BF16)],
        compiler_params=cparams,
    )(y1, st1, g1, be1, w2b)

    s = jnp.sum(st2[:, 0, :], axis=0)
    ss = jnp.sum(st2[:, 1, :], axis=0)
    mean = s / n_pix
    var = ss / n_pix - mean * mean
    scale = g2[0] * jax.lax.rsqrt(var + EPS)
    shift = be2[0] - mean * scale
    out = jnp.maximum(y2.astype(jnp.float32) * scale.reshape(1, 1, 1, Cout)
                      + shift.reshape(1, 1, 1, Cout), 0.0)
    return jnp.transpose(out, (0, 3, 1, 2))


# speedup vs baseline: 2.4880x; 1.4881x over previous
import jax
import jax.numpy as jnp
from jax.experimental import pallas as pl
from jax.experimental.pallas import tpu as pltpu

EPS = 1e-5
BF16 = jnp.bfloat16


def _zero_border(pad, nb, hp, wp, c):
    zrow = jnp.zeros((nb, 1, wp + 2, c), BF16)
    zcol = jnp.zeros((nb, hp + 2, 1, c), BF16)
    pad[:, 0:1, :, :] = zrow
    pad[:, hp + 1:hp + 2, :, :] = zrow
    pad[:, :, 0:1, :] = zcol
    pad[:, :, wp + 1:wp + 2, :] = zcol


def _conv3x3(pad, w_ref, nb, hp, wp, cin):
    cols = jnp.concatenate(
        [pad[:, dy:dy + hp, dx:dx + wp, :] for dy in range(3) for dx in range(3)],
        axis=-1)
    a = cols.reshape(nb * hp * wp, 9 * cin)
    return jnp.dot(a, w_ref[...], preferred_element_type=jnp.float32)


def _emit_stats(y, st_ref, cout):
    s = jnp.sum(y, axis=0, keepdims=True)
    ss = jnp.sum(y * y, axis=0, keepdims=True)
    st_ref[...] = jnp.concatenate([s, ss], axis=0).reshape(1, 2, cout)


def _stage1_kernel(x_ref, w_ref, y_ref, st_ref, rscr, pad):
    nb, h, w, c = x_ref.shape
    hp, wp = h // 2, w // 2
    cout = w_ref.shape[1]

    @pl.when(pl.program_id(0) == 0)
    def _():
        _zero_border(pad, nb, hp, wp, c)

    rows = jnp.maximum(x_ref[:, pl.Slice(0, hp, 2), :, :],
                       x_ref[:, pl.Slice(1, hp, 2), :, :])
    rscr[...] = rows
    pooled = jnp.maximum(rscr[:, :, pl.Slice(0, wp, 2), :],
                         rscr[:, :, pl.Slice(1, wp, 2), :])
    pad[:, 1:hp + 1, 1:wp + 1, :] = pooled.astype(BF16)
    y = _conv3x3(pad, w_ref, nb, hp, wp, c)
    _emit_stats(y, st_ref, cout)
    y_ref[...] = y.astype(BF16).reshape(nb, hp, wp, cout)


def _make_stage2_kernel(inv_npix):
    def _kernel(y1_ref, st1_ref, g_ref, be_ref, w_ref, y_ref, st_ref, pad):
        nb, hp, wp, c = y1_ref.shape
        cout = w_ref.shape[1]

        @pl.when(pl.program_id(0) == 0)
        def _():
            _zero_border(pad, nb, hp, wp, c)

        st = st1_ref[...]
        s = jnp.sum(st[:, 0:1, :], axis=0)
        ss = jnp.sum(st[:, 1:2, :], axis=0)
        mean = s * inv_npix
        var = ss * inv_npix - mean * mean
        scale = g_ref[...] * jax.lax.rsqrt(var + EPS)
        shift = be_ref[...] - mean * scale

        h = jnp.maximum(y1_ref[...].astype(jnp.float32) * scale.reshape(1, 1, 1, c)
                        + shift.reshape(1, 1, 1, c), 0.0)
        pad[:, 1:hp + 1, 1:wp + 1, :] = h.astype(BF16)
        y = _conv3x3(pad, w_ref, nb, hp, wp, c)
        _emit_stats(y, st_ref, cout)
        y_ref[...] = y.astype(BF16).reshape(nb, hp, wp, cout)
    return _kernel


def _pick_images_per_block(n, hp, wp, c):
    budget = 24 * 1024 * 1024
    nb = 1
    for d in range(1, n + 1):
        if n % d:
            continue
        m = d * hp * wp
        need = 2 * (9 * m * c + d * (hp + 2) * (wp + 2) * c + 4 * m * c) + 8 * m * c
        if need > budget:
            break
        nb = d
    return nb


def kernel(x_nchw, w1, b1, g1, be1, w2, b2, g2, be2):
    N, Cin, H, W = x_nchw.shape
    Hp, Wp = H // 2, W // 2
    Cout = w1.shape[1]
    n_pix = float(N * Hp * Wp)

    nb = _pick_images_per_block(N, Hp, Wp, Cout)
    n_blk = N // nb

    x5 = jnp.transpose(x_nchw[:, :, :2 * Hp, :2 * Wp], (0, 2, 3, 1))
    w1b = w1.astype(BF16)
    w2b = w2.astype(BF16)

    cparams = pltpu.CompilerParams(
        dimension_semantics=("arbitrary",),
        vmem_limit_bytes=48 * 1024 * 1024)

    y1, st1 = pl.pallas_call(
        _stage1_kernel,
        grid=(n_blk,),
        in_specs=[
            pl.BlockSpec((nb, 2 * Hp, 2 * Wp, Cin), lambda i: (i, 0, 0, 0)),
            pl.BlockSpec((9 * Cin, Cout), lambda i: (0, 0)),
        ],
        out_specs=(
            pl.BlockSpec((nb, Hp, Wp, Cout), lambda i: (i, 0, 0, 0)),
            pl.BlockSpec((1, 2, Cout), lambda i: (i, 0, 0)),
        ),
        out_shape=(
            jax.ShapeDtypeStruct((N, Hp, Wp, Cout), BF16),
            jax.ShapeDtypeStruct((n_blk, 2, Cout), jnp.float32),
        ),
        scratch_shapes=[pltpu.VMEM((nb, Hp, 2 * Wp, Cin), jnp.float32),
                        pltpu.VMEM((nb, Hp + 2, Wp + 2, Cin), BF16)],
        compiler_params=cparams,
    )(x5, w1b)

    y2, st2 = pl.pallas_call(
        _make_stage2_kernel(1.0 / n_pix),
        grid=(n_blk,),
        in_specs=[
            pl.BlockSpec((nb, Hp, Wp, Cout), lambda i: (i, 0, 0, 0)),
            pl.BlockSpec((n_blk, 2, Cout), lambda i: (0, 0, 0)),
            pl.BlockSpec((1, Cout), lambda i: (0, 0)),
            pl.BlockSpec((1, Cout), lambda i: (0, 0)),
            pl.BlockSpec((9 * Cout, Cout), lambda i: (0, 0)),
        ],
        out_specs=(
            pl.BlockSpec((nb, Hp, Wp, Cout), lambda i: (i, 0, 0, 0)),
            pl.BlockSpec((1, 2, Cout), lambda i: (i, 0, 0)),
        ),
        out_shape=(
            jax.ShapeDtypeStruct((N, Hp, Wp, Cout), BF16),
            jax.ShapeDtypeStruct((n_blk, 2, Cout), jnp.float32),
        ),
        scratch_shapes=[pltpu.VMEM((nb, Hp + 2, Wp + 2, Cout), BF16)],
        compiler_params=cparams,
    )(y1, st1, g1, be1, w2b)

    s = jnp.sum(st2[:, 0, :], axis=0)
    ss = jnp.sum(st2[:, 1, :], axis=0)
    mean = s / n_pix
    var = ss / n_pix - mean * mean
    scale = g2[0] * jax.lax.rsqrt(var + EPS)
    shift = be2[0] - mean * scale
    out = jnp.maximum(y2.astype(jnp.float32) * scale.reshape(1, 1, 1, Cout)
                      + shift.reshape(1, 1, 1, Cout), 0.0)
    return jnp.transpose(out, (0, 3, 1, 2))
